# baseline (speedup 1.0000x reference)
import functools

import jax
import jax.numpy as jnp
from jax import lax
from jax.experimental import pallas as pl
from jax.experimental.pallas import tpu as pltpu
from jax.experimental.pallas import tpu_sc as plsc

N_NODES = 10000
N_EDGES = 160000
D_NODE = 256
D_EDGE = 16
D_OUT = 256
HALF = 128
NC = 2
NS = 16
LANES = 16
K = 40
EPS = N_EDGES // NS
NCHUNK = EPS // K
NG = NCHUNK // 2
ROWS_MAIN = 640
GR = 80


def _mm_body(a_ref, w_ref, b_ref, o_ref):
    acc = jnp.dot(a_ref[...], w_ref[...], preferred_element_type=jnp.float32)
    b = b_ref[pl.ds(pl.program_id(0), 1), :]
    o_ref[...] = jnp.maximum(acc + b, 0.0)


def _linear_relu_split(a, W, b2, rows, rb):
    nrb = rows // rb
    return pl.pallas_call(
        _mm_body,
        grid=(NC, nrb),
        in_specs=[
            pl.BlockSpec((rb, a.shape[1]), lambda c, r: (r, 0)),
            pl.BlockSpec((a.shape[1], HALF), lambda c, r: (0, c)),
            pl.BlockSpec((NC, HALF), lambda c, r: (0, 0)),
        ],
        out_specs=pl.BlockSpec((rb, HALF), lambda c, r, nrb=nrb: (c * nrb + r, 0)),
        out_shape=jax.ShapeDtypeStruct((NC * rows, HALF), jnp.float32),
    )(a, W, b2)


def _sc_body(w_hbm, h_hbm, srci_hbm, dsti_hbm, dstf_hbm, out_hbm,
             srcb, dstb, dst80, wbuf, hbuf, obuf, degv, ones_v,
             acc, dacc, sem_i, sem_g, sem_h, sem_s, sem_d):
    c = lax.axis_index("c")
    s = lax.axis_index("s")
    r0 = s * ROWS_MAIN
    ngroups = jnp.where(s < NS - 1, ROWS_MAIN // GR,
                        (N_NODES - (NS - 1) * ROWS_MAIN) // GR)
    rowbase = (c * NS + s) * NCHUNK
    hbase = c * N_EDGES + s * EPS
    ebase = c * N_EDGES + s * EPS

    zero16 = jnp.zeros((LANES,), jnp.float32)
    one16 = jnp.ones((LANES,), jnp.float32)

    @pl.loop(0, GR)
    def _(k):
        for j in range(HALF // LANES):
            obuf[k, pl.ds(j * LANES, LANES)] = zero16

    @pl.loop(0, GR // LANES)
    def _(q):
        ones_v[pl.ds(q * LANES, LANES)] = one16
        degv[pl.ds(q * LANES, LANES)] = zero16

    @pl.loop(0, ngroups)
    def _(g):
        rb = r0 + g * GR
        pltpu.sync_copy(obuf, acc.at[pl.ds(rb, GR)])
        pltpu.sync_copy(degv, dacc.at[pl.ds(rb, GR)])

    plsc.subcore_barrier()

    def issue_idx(m):
        p = lax.rem(m, 2)
        pltpu.async_copy(srci_hbm.at[pl.ds(rowbase + 2 * m, 2)],
                         srcb.at[p], sem_i.at[p])
        pltpu.async_copy(dsti_hbm.at[pl.ds(rowbase + 2 * m, 2)],
                         dstb.at[p], sem_i.at[p])
        pltpu.async_copy(dstf_hbm.at[pl.ds(ebase + m * 2 * K, 2 * K)],
                         dst80.at[p], sem_i.at[p])

    def wait_idx(m):
        p = lax.rem(m, 2)
        pltpu.make_async_copy(srci_hbm.at[pl.ds(rowbase, 2)],
                              srcb.at[p], sem_i.at[p]).wait()
        pltpu.make_async_copy(dsti_hbm.at[pl.ds(rowbase, 2)],
                              dstb.at[p], sem_i.at[p]).wait()
        pltpu.make_async_copy(dstf_hbm.at[pl.ds(ebase, 2 * K)],
                              dst80.at[p], sem_i.at[p]).wait()

    def issue_loads(i, p, sub):
        pltpu.async_copy(w_hbm.at[srcb.at[p, sub]], wbuf.at[sub],
                         sem_g.at[sub])
        pltpu.async_copy(h_hbm.at[pl.ds(hbase + i * K, K)], hbuf.at[sub],
                         sem_h.at[sub])

    def wait_loads(sub):
        pltpu.make_async_copy(w_hbm.at[srcb.at[0, 0]], wbuf.at[sub],
                              sem_g.at[sub]).wait()
        pltpu.make_async_copy(h_hbm.at[pl.ds(hbase, K)], hbuf.at[sub],
                              sem_h.at[sub]).wait()

    def mult(sub):
        @pl.loop(0, K, unroll=4)
        def _(k):
            for j in range(HALF // LANES):
                sl = (sub, k, pl.ds(j * LANES, LANES))
                wbuf[sl] = wbuf[sl] * hbuf[sl]

    def issue_scatter(p, sub):
        pltpu.async_copy(wbuf.at[sub], acc.at[dstb.at[p, sub]],
                         sem_s.at[sub], add=True)

    def wait_scatter(sub):
        pltpu.make_async_copy(wbuf.at[sub], acc.at[dstb.at[0, 0]],
                              sem_s.at[sub]).wait()

    def wait_deg():
        pltpu.make_async_copy(ones_v, dacc.at[dst80.at[0]], sem_d).wait()

    pltpu.sync_copy(srci_hbm.at[pl.ds(rowbase, 2)], srcb.at[0])
    pltpu.sync_copy(dsti_hbm.at[pl.ds(rowbase, 2)], dstb.at[0])
    pltpu.sync_copy(dstf_hbm.at[pl.ds(ebase, 2 * K)], dst80.at[0])
    issue_loads(0, 0, 0)

    @pl.loop(0, NG)
    def _(m):
        p = lax.rem(m, 2)
        np_ = 1 - p

        @pl.when(m >= 1)
        def _():
            wait_scatter(1)
        issue_loads(2 * m + 1, p, 1)

        @pl.when(m >= 1)
        def _():
            wait_deg()
        @pl.when(m + 1 < NG)
        def _():
            issue_idx(m + 1)
        pltpu.async_copy(ones_v, dacc.at[dst80.at[p]], sem_d, add=True)

        wait_loads(0)
        mult(0)
        issue_scatter(p, 0)

        wait_loads(1)
        mult(1)
        issue_scatter(p, 1)

        @pl.when(m + 1 < NG)
        def _():
            wait_scatter(0)
            wait_idx(m + 1)
            issue_loads(2 * (m + 1), 1 - lax.rem(m, 2), 0)

    wait_scatter(0)
    wait_scatter(1)
    wait_deg()

    plsc.subcore_barrier()

    @pl.loop(0, ngroups)
    def _(g):
        rb = r0 + g * GR
        pltpu.sync_copy(acc.at[pl.ds(rb, GR)], obuf)
        pltpu.sync_copy(dacc.at[pl.ds(rb, GR)], degv)

        @pl.loop(0, GR // LANES)
        def _(q):
            d = degv[pl.ds(q * LANES, LANES)]
            inv = 1.0 / jnp.maximum(d, 1.0)
            for kk in range(LANES):
                bc = jnp.take_along_axis(
                    inv, jnp.full((LANES,), kk, jnp.int32), axis=0
                )
                row = q * LANES + kk
                for j in range(HALF // LANES):
                    sl = (row, pl.ds(j * LANES, LANES))
                    obuf[sl] = obuf[sl] * bc

        pltpu.sync_copy(obuf,
                        out_hbm.at[pl.ds(rb, GR), pl.ds(c * HALF, HALF)])


_sc_call = pl.kernel(
    _sc_body,
    out_type=jax.ShapeDtypeStruct((N_NODES, D_OUT), jnp.float32),
    mesh=plsc.VectorSubcoreMesh(
        core_axis_name="c", subcore_axis_name="s", num_cores=NC, num_subcores=NS
    ),
    compiler_params=pltpu.CompilerParams(use_tc_tiling_on_sc=False),
    scratch_types=[
        pltpu.VMEM((2, 2, K), jnp.int32),
        pltpu.VMEM((2, 2, K), jnp.int32),
        pltpu.VMEM((2, 2 * K), jnp.int32),
        pltpu.VMEM((2, K, HALF), jnp.float32),
        pltpu.VMEM((2, K, HALF), jnp.float32),
        pltpu.VMEM((GR, HALF), jnp.float32),
        pltpu.VMEM((GR,), jnp.float32),
        pltpu.VMEM((GR,), jnp.float32),
        pltpu.VMEM_SHARED((N_NODES, HALF), jnp.float32),
        pltpu.VMEM_SHARED((N_NODES,), jnp.float32),
        pltpu.SemaphoreType.DMA((2,)),
        pltpu.SemaphoreType.DMA((2,)),
        pltpu.SemaphoreType.DMA((2,)),
        pltpu.SemaphoreType.DMA((2,)),
        pltpu.SemaphoreType.DMA,
    ],
)


def kernel(x, edge_attr, edge_index, Wn, bn, We, be):
    ei = edge_index.astype(jnp.int32)
    src, dst = ei[0], ei[1]
    srci = jnp.stack([src, src + N_NODES]).reshape(NC * N_EDGES // K, K)
    dstf = jnp.broadcast_to(dst, (NC, N_EDGES))
    dsti = dstf.reshape(NC * N_EDGES // K, K)
    dstf = dstf.reshape(NC * N_EDGES)
    w_cat = _linear_relu_split(x, Wn, bn.reshape(NC, HALF), N_NODES, 2000)
    h_cat = _linear_relu_split(edge_attr, We, be.reshape(NC, HALF), N_EDGES, 8000)
    return _sc_call(w_cat, h_cat, srci, dsti, dstf)


# speedup vs baseline: 1.5652x; 1.5652x over previous
import functools

import jax
import jax.numpy as jnp
from jax import lax
from jax.experimental import pallas as pl
from jax.experimental.pallas import tpu as pltpu
from jax.experimental.pallas import tpu_sc as plsc

N_NODES = 10000
N_EDGES = 160000
D_NODE = 256
D_EDGE = 16
D_OUT = 256
HALF = 128
NC = 2
NS = 16
LANES = 16
K = 40
EPS = N_EDGES // NS
NCHUNK = EPS // K
NG = NCHUNK // 2
ROWS_MAIN = 640
GR = 80


def _mm_body(a_ref, w_ref, b_ref, o_ref):
    acc = jnp.dot(a_ref[...], w_ref[...], preferred_element_type=jnp.float32)
    b = b_ref[pl.ds(pl.program_id(0), 1), :]
    o_ref[...] = jnp.maximum(acc + b, 0.0).astype(o_ref.dtype)


def _linear_relu_split(a, W, b2, rows, rb, out_dtype=jnp.float32):
    nrb = rows // rb
    return pl.pallas_call(
        _mm_body,
        grid=(NC, nrb),
        in_specs=[
            pl.BlockSpec((rb, a.shape[1]), lambda c, r: (r, 0)),
            pl.BlockSpec((a.shape[1], HALF), lambda c, r: (0, c)),
            pl.BlockSpec((NC, HALF), lambda c, r: (0, 0)),
        ],
        out_specs=pl.BlockSpec((rb, HALF), lambda c, r, nrb=nrb: (c * nrb + r, 0)),
        out_shape=jax.ShapeDtypeStruct((NC * rows, HALF), out_dtype),
    )(a, W, b2)


def _sc_body(w_hbm, h_hbm, srci_hbm, dsti_hbm, dstf_hbm, out_hbm,
             srcb, dstb, dst80, wbuf, hbuf, obuf, degv, ones_v,
             acc, dacc, sem_i, sem_g, sem_h, sem_s, sem_d):
    c = lax.axis_index("c")
    s = lax.axis_index("s")
    r0 = s * ROWS_MAIN
    ngroups = jnp.where(s < NS - 1, ROWS_MAIN // GR,
                        (N_NODES - (NS - 1) * ROWS_MAIN) // GR)
    rowbase = (c * NS + s) * NCHUNK
    hbase = c * N_EDGES + s * EPS
    ebase = c * N_EDGES + s * EPS

    zero16 = jnp.zeros((LANES,), jnp.float32)
    one16 = jnp.ones((LANES,), jnp.float32)

    @pl.loop(0, GR)
    def _(k):
        for j in range(HALF // LANES):
            obuf[k, pl.ds(j * LANES, LANES)] = zero16

    @pl.loop(0, GR // LANES)
    def _(q):
        ones_v[pl.ds(q * LANES, LANES)] = one16
        degv[pl.ds(q * LANES, LANES)] = zero16

    @pl.loop(0, ngroups)
    def _(g):
        rb = r0 + g * GR
        pltpu.sync_copy(obuf, acc.at[pl.ds(rb, GR)])
        pltpu.sync_copy(degv, dacc.at[pl.ds(rb, GR)])

    plsc.subcore_barrier()

    def issue_idx(m):
        p = lax.rem(m, 2)
        pltpu.async_copy(srci_hbm.at[pl.ds(rowbase + 2 * m, 2)],
                         srcb.at[p], sem_i.at[p])
        pltpu.async_copy(dsti_hbm.at[pl.ds(rowbase + 2 * m, 2)],
                         dstb.at[p], sem_i.at[p])
        pltpu.async_copy(dstf_hbm.at[pl.ds(ebase + m * 2 * K, 2 * K)],
                         dst80.at[p], sem_i.at[p])

    def wait_idx(m):
        p = lax.rem(m, 2)
        pltpu.make_async_copy(srci_hbm.at[pl.ds(rowbase, 2)],
                              srcb.at[p], sem_i.at[p]).wait()
        pltpu.make_async_copy(dsti_hbm.at[pl.ds(rowbase, 2)],
                              dstb.at[p], sem_i.at[p]).wait()
        pltpu.make_async_copy(dstf_hbm.at[pl.ds(ebase, 2 * K)],
                              dst80.at[p], sem_i.at[p]).wait()

    def issue_loads(i, p, sub):
        pltpu.async_copy(w_hbm.at[srcb.at[p, sub]], wbuf.at[sub],
                         sem_g.at[sub])
        pltpu.async_copy(h_hbm.at[pl.ds(hbase + i * K, K)], hbuf.at[sub],
                         sem_h.at[sub])

    def wait_loads(sub):
        pltpu.make_async_copy(w_hbm.at[srcb.at[0, 0]], wbuf.at[sub],
                              sem_g.at[sub]).wait()
        pltpu.make_async_copy(h_hbm.at[pl.ds(hbase, K)], hbuf.at[sub],
                              sem_h.at[sub]).wait()

    def mult(sub):
        @pl.loop(0, K)
        def _(k):
            for j in range(HALF // LANES):
                sl = (sub, k, pl.ds(j * LANES, LANES))
                wbuf[sl] = wbuf[sl] * hbuf[sl]

    def issue_scatter(p, sub):
        pltpu.async_copy(wbuf.at[sub], acc.at[dstb.at[p, sub]],
                         sem_s.at[sub], add=True)

    def wait_scatter(sub):
        pltpu.make_async_copy(wbuf.at[sub], acc.at[dstb.at[0, 0]],
                              sem_s.at[sub]).wait()

    def wait_deg():
        pltpu.make_async_copy(ones_v, dacc.at[dst80.at[0]], sem_d).wait()

    pltpu.sync_copy(srci_hbm.at[pl.ds(rowbase, 2)], srcb.at[0])
    pltpu.sync_copy(dsti_hbm.at[pl.ds(rowbase, 2)], dstb.at[0])
    pltpu.sync_copy(dstf_hbm.at[pl.ds(ebase, 2 * K)], dst80.at[0])
    issue_loads(0, 0, 0)

    @pl.loop(0, NG)
    def _(m):
        p = lax.rem(m, 2)
        np_ = 1 - p

        @pl.when(m >= 1)
        def _():
            wait_scatter(1)
        issue_loads(2 * m + 1, p, 1)

        @pl.when(m >= 1)
        def _():
            wait_deg()
        @pl.when(m + 1 < NG)
        def _():
            issue_idx(m + 1)
        pltpu.async_copy(ones_v, dacc.at[dst80.at[p]], sem_d, add=True)

        wait_loads(0)
        mult(0)
        issue_scatter(p, 0)

        wait_loads(1)
        mult(1)
        issue_scatter(p, 1)

        @pl.when(m + 1 < NG)
        def _():
            wait_scatter(0)
            wait_idx(m + 1)
            issue_loads(2 * (m + 1), 1 - lax.rem(m, 2), 0)

    wait_scatter(0)
    wait_scatter(1)
    wait_deg()

    plsc.subcore_barrier()

    @pl.loop(0, ngroups)
    def _(g):
        rb = r0 + g * GR
        pltpu.sync_copy(acc.at[pl.ds(rb, GR)], obuf)
        pltpu.sync_copy(dacc.at[pl.ds(rb, GR)], degv)

        @pl.loop(0, GR // LANES)
        def _(q):
            d = degv[pl.ds(q * LANES, LANES)]
            inv = 1.0 / jnp.maximum(d, 1.0)
            for kk in range(LANES):
                bc = jnp.take_along_axis(
                    inv, jnp.full((LANES,), kk, jnp.int32), axis=0
                )
                row = q * LANES + kk
                for j in range(HALF // LANES):
                    sl = (row, pl.ds(j * LANES, LANES))
                    obuf[sl] = obuf[sl] * bc

        pltpu.sync_copy(obuf,
                        out_hbm.at[pl.ds(rb, GR), pl.ds(c * HALF, HALF)])


_sc_call = pl.kernel(
    _sc_body,
    out_type=jax.ShapeDtypeStruct((N_NODES, D_OUT), jnp.float32),
    mesh=plsc.VectorSubcoreMesh(
        core_axis_name="c", subcore_axis_name="s", num_cores=NC, num_subcores=NS
    ),
    compiler_params=pltpu.CompilerParams(use_tc_tiling_on_sc=False),
    scratch_types=[
        pltpu.VMEM((2, 2, K), jnp.int32),
        pltpu.VMEM((2, 2, K), jnp.int32),
        pltpu.VMEM((2, 2 * K), jnp.int32),
        pltpu.VMEM((2, K, HALF), jnp.float32),
        pltpu.VMEM((2, K, HALF), jnp.float32),
        pltpu.VMEM((GR, HALF), jnp.float32),
        pltpu.VMEM((GR,), jnp.float32),
        pltpu.VMEM((GR,), jnp.float32),
        pltpu.VMEM_SHARED((N_NODES, HALF), jnp.float32),
        pltpu.VMEM_SHARED((N_NODES,), jnp.float32),
        pltpu.SemaphoreType.DMA((2,)),
        pltpu.SemaphoreType.DMA((2,)),
        pltpu.SemaphoreType.DMA((2,)),
        pltpu.SemaphoreType.DMA((2,)),
        pltpu.SemaphoreType.DMA,
    ],
)


def kernel(x, edge_attr, edge_index, Wn, bn, We, be):
    ei = edge_index.astype(jnp.int32)
    src, dst = ei[0], ei[1]
    srci = jnp.stack([src, src + N_NODES]).reshape(NC * N_EDGES // K, K)
    dstf = jnp.broadcast_to(dst, (NC, N_EDGES))
    dsti = dstf.reshape(NC * N_EDGES // K, K)
    dstf = dstf.reshape(NC * N_EDGES)
    w_cat = _linear_relu_split(x, Wn, bn.reshape(NC, HALF), N_NODES, 2000)
    h_cat = _linear_relu_split(edge_attr, We, be.reshape(NC, HALF), N_EDGES, 16000)
    return _sc_call(w_cat, h_cat, srci, dsti, dstf)


# speedup vs baseline: 1.5836x; 1.0117x over previous
import functools

import jax
import jax.numpy as jnp
from jax import lax
from jax.experimental import pallas as pl
from jax.experimental.pallas import tpu as pltpu
from jax.experimental.pallas import tpu_sc as plsc

N_NODES = 10000
N_EDGES = 160000
D_NODE = 256
D_EDGE = 16
D_OUT = 256
HALF = 128
NC = 2
NS = 16
LANES = 16
K = 40
EPS = N_EDGES // NS
NCHUNK = EPS // K
NG = NCHUNK // 2
ROWS_MAIN = 640
GR = 80


def _mm_body(a_ref, w_ref, b_ref, o_ref):
    acc = jnp.dot(a_ref[...], w_ref[...], preferred_element_type=jnp.float32)
    b = b_ref[pl.ds(pl.program_id(0), 1), :]
    o_ref[...] = jnp.maximum(acc + b, 0.0).astype(o_ref.dtype)


def _linear_relu_split(a, W, b2, rows, rb, out_dtype=jnp.float32):
    nrb = rows // rb
    return pl.pallas_call(
        _mm_body,
        grid=(NC, nrb),
        in_specs=[
            pl.BlockSpec((rb, a.shape[1]), lambda c, r: (r, 0)),
            pl.BlockSpec((a.shape[1], HALF), lambda c, r: (0, c)),
            pl.BlockSpec((NC, HALF), lambda c, r: (0, 0)),
        ],
        out_specs=pl.BlockSpec((rb, HALF), lambda c, r, nrb=nrb: (c * nrb + r, 0)),
        out_shape=jax.ShapeDtypeStruct((NC * rows, HALF), out_dtype),
    )(a, W, b2)


def _sc_body(w_hbm, h_hbm, srci_hbm, dsti_hbm, dstf_hbm, out_hbm,
             srcb, dstb, dst80, wbuf, hbuf, obuf, degv, ones_v,
             acc, dacc, sem_i, sem_g, sem_h, sem_s, sem_d):
    c = lax.axis_index("c")
    s = lax.axis_index("s")
    r0 = s * ROWS_MAIN
    ngroups = jnp.where(s < NS - 1, ROWS_MAIN // GR,
                        (N_NODES - (NS - 1) * ROWS_MAIN) // GR)
    rowbase = (c * NS + s) * NCHUNK
    hbase = c * N_EDGES + s * EPS
    ebase = c * N_EDGES + s * EPS

    zero16 = jnp.zeros((LANES,), jnp.float32)
    one16 = jnp.ones((LANES,), jnp.float32)

    @pl.loop(0, GR)
    def _(k):
        for j in range(HALF // LANES):
            obuf[k, pl.ds(j * LANES, LANES)] = zero16

    @pl.loop(0, GR // LANES)
    def _(q):
        ones_v[pl.ds(q * LANES, LANES)] = one16
        degv[pl.ds(q * LANES, LANES)] = zero16

    @pl.loop(0, ngroups)
    def _(g):
        rb = r0 + g * GR
        pltpu.sync_copy(obuf, acc.at[pl.ds(rb, GR)])
        pltpu.sync_copy(degv, dacc.at[pl.ds(rb, GR)])

    plsc.subcore_barrier()

    def issue_idx(m):
        p = lax.rem(m, 2)
        pltpu.async_copy(srci_hbm.at[pl.ds(rowbase + 2 * m, 2)],
                         srcb.at[p], sem_i.at[p])
        pltpu.async_copy(dsti_hbm.at[pl.ds(rowbase + 2 * m, 2)],
                         dstb.at[p], sem_i.at[p])
        pltpu.async_copy(dstf_hbm.at[pl.ds(ebase + m * 2 * K, 2 * K)],
                         dst80.at[p], sem_i.at[p])

    def wait_idx(m):
        p = lax.rem(m, 2)
        pltpu.make_async_copy(srci_hbm.at[pl.ds(rowbase, 2)],
                              srcb.at[p], sem_i.at[p]).wait()
        pltpu.make_async_copy(dsti_hbm.at[pl.ds(rowbase, 2)],
                              dstb.at[p], sem_i.at[p]).wait()
        pltpu.make_async_copy(dstf_hbm.at[pl.ds(ebase, 2 * K)],
                              dst80.at[p], sem_i.at[p]).wait()

    def issue_loads(i, p, sub):
        pltpu.async_copy(w_hbm.at[srcb.at[p, sub]], wbuf.at[sub],
                         sem_g.at[sub])
        pltpu.async_copy(h_hbm.at[pl.ds(hbase + i * K, K)], hbuf.at[sub],
                         sem_h.at[sub])

    def wait_loads(sub):
        pltpu.make_async_copy(w_hbm.at[srcb.at[0, 0]], wbuf.at[sub],
                              sem_g.at[sub]).wait()
        pltpu.make_async_copy(h_hbm.at[pl.ds(hbase, K)], hbuf.at[sub],
                              sem_h.at[sub]).wait()

    def mult(sub):
        @pl.loop(0, K)
        def _(k):
            for j in range(HALF // LANES):
                sl = (sub, k, pl.ds(j * LANES, LANES))
                wbuf[sl] = wbuf[sl] * hbuf[sl]

    def issue_scatter(p, sub):
        pltpu.async_copy(wbuf.at[sub], acc.at[dstb.at[p, sub]],
                         sem_s.at[sub], add=True)

    def wait_scatter(sub):
        pltpu.make_async_copy(wbuf.at[sub], acc.at[dstb.at[0, 0]],
                              sem_s.at[sub]).wait()

    def wait_deg():
        pltpu.make_async_copy(ones_v, dacc.at[dst80.at[0]], sem_d).wait()

    pltpu.sync_copy(srci_hbm.at[pl.ds(rowbase, 2)], srcb.at[0])
    pltpu.sync_copy(dsti_hbm.at[pl.ds(rowbase, 2)], dstb.at[0])
    pltpu.sync_copy(dstf_hbm.at[pl.ds(ebase, 2 * K)], dst80.at[0])
    issue_loads(0, 0, 0)

    @pl.loop(0, NG)
    def _(m):
        p = lax.rem(m, 2)
        np_ = 1 - p

        @pl.when(m >= 1)
        def _():
            wait_scatter(1)
        issue_loads(2 * m + 1, p, 1)

        @pl.when(m >= 1)
        def _():
            wait_deg()
        @pl.when(m + 1 < NG)
        def _():
            issue_idx(m + 1)
        pltpu.async_copy(ones_v, dacc.at[dst80.at[p]], sem_d, add=True)

        wait_loads(0)
        mult(0)
        issue_scatter(p, 0)

        wait_loads(1)
        mult(1)
        issue_scatter(p, 1)

        @pl.when(m + 1 < NG)
        def _():
            wait_scatter(0)
            wait_idx(m + 1)
            issue_loads(2 * (m + 1), 1 - lax.rem(m, 2), 0)

    wait_scatter(0)
    wait_scatter(1)
    wait_deg()

    plsc.subcore_barrier()

    @pl.loop(0, ngroups)
    def _(g):
        rb = r0 + g * GR
        pltpu.sync_copy(acc.at[pl.ds(rb, GR)], obuf)
        pltpu.sync_copy(dacc.at[pl.ds(rb, GR)], degv)

        @pl.loop(0, GR // LANES)
        def _(q):
            d = degv[pl.ds(q * LANES, LANES)]
            inv = 1.0 / jnp.maximum(d, 1.0)
            for kk in range(LANES):
                bc = jnp.take_along_axis(
                    inv, jnp.full((LANES,), kk, jnp.int32), axis=0
                )
                row = q * LANES + kk
                for j in range(HALF // LANES):
                    sl = (row, pl.ds(j * LANES, LANES))
                    obuf[sl] = obuf[sl] * bc

        pltpu.sync_copy(obuf,
                        out_hbm.at[pl.ds(rb, GR), pl.ds(c * HALF, HALF)])


_sc_call = pl.kernel(
    _sc_body,
    out_type=jax.ShapeDtypeStruct((N_NODES, D_OUT), jnp.float32),
    mesh=plsc.VectorSubcoreMesh(
        core_axis_name="c", subcore_axis_name="s", num_cores=NC, num_subcores=NS
    ),
    compiler_params=pltpu.CompilerParams(use_tc_tiling_on_sc=False),
    scratch_types=[
        pltpu.VMEM((2, 2, K), jnp.int32),
        pltpu.VMEM((2, 2, K), jnp.int32),
        pltpu.VMEM((2, 2 * K), jnp.int32),
        pltpu.VMEM((2, K, HALF), jnp.float32),
        pltpu.VMEM((2, K, HALF), jnp.float32),
        pltpu.VMEM((GR, HALF), jnp.float32),
        pltpu.VMEM((GR,), jnp.float32),
        pltpu.VMEM((GR,), jnp.float32),
        pltpu.VMEM_SHARED((N_NODES, HALF), jnp.float32),
        pltpu.VMEM_SHARED((N_NODES,), jnp.float32),
        pltpu.SemaphoreType.DMA((2,)),
        pltpu.SemaphoreType.DMA((2,)),
        pltpu.SemaphoreType.DMA((2,)),
        pltpu.SemaphoreType.DMA((2,)),
        pltpu.SemaphoreType.DMA,
    ],
)


def kernel(x, edge_attr, edge_index, Wn, bn, We, be):
    ei = edge_index.astype(jnp.int32)
    src, dst = ei[0], ei[1]
    srci = jnp.stack([src, src + N_NODES]).reshape(NC * N_EDGES // K, K)
    dstf = jnp.broadcast_to(dst, (NC, N_EDGES))
    dsti = dstf.reshape(NC * N_EDGES // K, K)
    dstf = dstf.reshape(NC * N_EDGES)
    w_cat = _linear_relu_split(x, Wn, bn.reshape(NC, HALF), N_NODES, 10000)
    h_cat = _linear_relu_split(edge_attr, We, be.reshape(NC, HALF), N_EDGES, 16000)
    return _sc_call(w_cat, h_cat, srci, dsti, dstf)


# speedup vs baseline: 1.7264x; 1.0902x over previous
import functools

import jax
import jax.numpy as jnp
from jax import lax
from jax.experimental import pallas as pl
from jax.experimental.pallas import tpu as pltpu
from jax.experimental.pallas import tpu_sc as plsc

N_NODES = 10000
N_EDGES = 160000
D_NODE = 256
D_EDGE = 16
D_OUT = 256
HALF = 128
NC = 2
NS = 16
LANES = 16
K = 40
EPS = N_EDGES // NS
NCHUNK = EPS // K
NG = NCHUNK // 2
ROWS_MAIN = 640
GR = 80


def _mm_body(a_ref, w_ref, b_ref, o_ref):
    acc = jnp.dot(a_ref[...], w_ref[...], preferred_element_type=jnp.float32)
    b = b_ref[pl.ds(pl.program_id(0), 1), :]
    o_ref[...] = jnp.maximum(acc + b, 0.0).astype(o_ref.dtype)


def _linear_relu_split(a, W, b2, rows, rb, out_dtype=jnp.float32):
    nrb = rows // rb
    return pl.pallas_call(
        _mm_body,
        grid=(NC, nrb),
        in_specs=[
            pl.BlockSpec((rb, a.shape[1]), lambda c, r: (r, 0)),
            pl.BlockSpec((a.shape[1], HALF), lambda c, r: (0, c)),
            pl.BlockSpec((NC, HALF), lambda c, r: (0, 0)),
        ],
        out_specs=pl.BlockSpec((rb, HALF), lambda c, r, nrb=nrb: (c * nrb + r, 0)),
        out_shape=jax.ShapeDtypeStruct((NC * rows, HALF), out_dtype),
    )(a, W, b2)


def _sc_body(w_hbm, h_hbm, srci_hbm, dsti_hbm, dstf_hbm, out_hbm,
             srcb, dstb, dst80, wbuf, hbuf, obuf, degv, ones_v,
             acc, dacc, sem_i, sem_g, sem_h, sem_s, sem_d):
    c = lax.axis_index("c")
    s = lax.axis_index("s")
    r0 = s * ROWS_MAIN
    ngroups = jnp.where(s < NS - 1, ROWS_MAIN // GR,
                        (N_NODES - (NS - 1) * ROWS_MAIN) // GR)
    rowbase = (c * NS + s) * NCHUNK
    hbase = c * N_EDGES + s * EPS
    ebase = c * N_EDGES + s * EPS

    zero16 = jnp.zeros((LANES,), jnp.float32)
    one16 = jnp.ones((LANES,), jnp.float32)

    @pl.loop(0, GR)
    def _(k):
        for j in range(HALF // LANES):
            obuf[k, pl.ds(j * LANES, LANES)] = zero16

    @pl.loop(0, GR // LANES)
    def _(q):
        ones_v[pl.ds(q * LANES, LANES)] = one16
        degv[pl.ds(q * LANES, LANES)] = zero16

    @pl.loop(0, ngroups)
    def _(g):
        rb = r0 + g * GR
        pltpu.sync_copy(obuf, acc.at[pl.ds(rb, GR)])
        pltpu.sync_copy(degv, dacc.at[pl.ds(rb, GR)])

    plsc.subcore_barrier()

    def issue_idx(m):
        p = lax.rem(m, 2)
        pltpu.async_copy(srci_hbm.at[pl.ds(rowbase + 2 * m, 2)],
                         srcb.at[p], sem_i.at[p])
        pltpu.async_copy(dsti_hbm.at[pl.ds(rowbase + 2 * m, 2)],
                         dstb.at[p], sem_i.at[p])
        pltpu.async_copy(dstf_hbm.at[pl.ds(ebase + m * 2 * K, 2 * K)],
                         dst80.at[p], sem_i.at[p])

    def wait_idx(m):
        p = lax.rem(m, 2)
        pltpu.make_async_copy(srci_hbm.at[pl.ds(rowbase, 2)],
                              srcb.at[p], sem_i.at[p]).wait()
        pltpu.make_async_copy(dsti_hbm.at[pl.ds(rowbase, 2)],
                              dstb.at[p], sem_i.at[p]).wait()
        pltpu.make_async_copy(dstf_hbm.at[pl.ds(ebase, 2 * K)],
                              dst80.at[p], sem_i.at[p]).wait()

    def issue_loads(i, p, sub):
        pltpu.async_copy(w_hbm.at[srcb.at[p, sub]], wbuf.at[sub],
                         sem_g.at[sub])
        pltpu.async_copy(h_hbm.at[pl.ds(hbase + i * K, K)], hbuf.at[sub],
                         sem_h.at[sub])

    def wait_loads(sub):
        pltpu.make_async_copy(w_hbm.at[srcb.at[0, 0]], wbuf.at[sub],
                              sem_g.at[sub]).wait()
        pltpu.make_async_copy(h_hbm.at[pl.ds(hbase, K)], hbuf.at[sub],
                              sem_h.at[sub]).wait()

    def mult(sub):
        @pl.loop(0, K)
        def _(k):
            for j in range(HALF // LANES):
                sl = (sub, k, pl.ds(j * LANES, LANES))
                wbuf[sl] = wbuf[sl] * hbuf[sl]

    def issue_scatter(p, sub):
        pltpu.async_copy(wbuf.at[sub], acc.at[dstb.at[p, sub]],
                         sem_s.at[sub], add=True)

    def wait_scatter(sub):
        pltpu.make_async_copy(wbuf.at[sub], acc.at[dstb.at[0, 0]],
                              sem_s.at[sub]).wait()

    def wait_deg():
        pltpu.make_async_copy(ones_v, dacc.at[dst80.at[0]], sem_d).wait()

    pltpu.sync_copy(srci_hbm.at[pl.ds(rowbase, 2)], srcb.at[0])
    pltpu.sync_copy(dsti_hbm.at[pl.ds(rowbase, 2)], dstb.at[0])
    pltpu.sync_copy(dstf_hbm.at[pl.ds(ebase, 2 * K)], dst80.at[0])
    issue_loads(0, 0, 0)

    @pl.loop(0, NG)
    def _(m):
        p = lax.rem(m, 2)
        np_ = 1 - p

        @pl.when(m >= 1)
        def _():
            wait_scatter(1)
        issue_loads(2 * m + 1, p, 1)

        @pl.when(m >= 1)
        def _():
            wait_deg()
        @pl.when(m + 1 < NG)
        def _():
            issue_idx(m + 1)
        pltpu.async_copy(ones_v, dacc.at[dst80.at[p]], sem_d, add=True)

        wait_loads(0)
        mult(0)
        issue_scatter(p, 0)

        @pl.when(m + 1 < NG)
        def _():
            wait_scatter(0)
            wait_idx(m + 1)
            issue_loads(2 * (m + 1), 1 - lax.rem(m, 2), 0)

        wait_loads(1)
        mult(1)
        issue_scatter(p, 1)

    wait_scatter(0)
    wait_scatter(1)
    wait_deg()

    plsc.subcore_barrier()

    @pl.loop(0, ngroups)
    def _(g):
        rb = r0 + g * GR
        pltpu.sync_copy(acc.at[pl.ds(rb, GR)], obuf)
        pltpu.sync_copy(dacc.at[pl.ds(rb, GR)], degv)

        @pl.loop(0, GR // LANES)
        def _(q):
            d = degv[pl.ds(q * LANES, LANES)]
            inv = 1.0 / jnp.maximum(d, 1.0)
            for kk in range(LANES):
                bc = jnp.take_along_axis(
                    inv, jnp.full((LANES,), kk, jnp.int32), axis=0
                )
                row = q * LANES + kk
                for j in range(HALF // LANES):
                    sl = (row, pl.ds(j * LANES, LANES))
                    obuf[sl] = obuf[sl] * bc

        pltpu.sync_copy(obuf,
                        out_hbm.at[pl.ds(rb, GR), pl.ds(c * HALF, HALF)])


_sc_call = pl.kernel(
    _sc_body,
    out_type=jax.ShapeDtypeStruct((N_NODES, D_OUT), jnp.float32),
    mesh=plsc.VectorSubcoreMesh(
        core_axis_name="c", subcore_axis_name="s", num_cores=NC, num_subcores=NS
    ),
    compiler_params=pltpu.CompilerParams(use_tc_tiling_on_sc=False),
    scratch_types=[
        pltpu.VMEM((2, 2, K), jnp.int32),
        pltpu.VMEM((2, 2, K), jnp.int32),
        pltpu.VMEM((2, 2 * K), jnp.int32),
        pltpu.VMEM((2, K, HALF), jnp.float32),
        pltpu.VMEM((2, K, HALF), jnp.float32),
        pltpu.VMEM((GR, HALF), jnp.float32),
        pltpu.VMEM((GR,), jnp.float32),
        pltpu.VMEM((GR,), jnp.float32),
        pltpu.VMEM_SHARED((N_NODES, HALF), jnp.float32),
        pltpu.VMEM_SHARED((N_NODES,), jnp.float32),
        pltpu.SemaphoreType.DMA((2,)),
        pltpu.SemaphoreType.DMA((2,)),
        pltpu.SemaphoreType.DMA((2,)),
        pltpu.SemaphoreType.DMA((2,)),
        pltpu.SemaphoreType.DMA,
    ],
)


def kernel(x, edge_attr, edge_index, Wn, bn, We, be):
    ei = edge_index.astype(jnp.int32)
    src, dst = ei[0], ei[1]
    srci = jnp.stack([src, src + N_NODES]).reshape(NC * N_EDGES // K, K)
    dstf = jnp.broadcast_to(dst, (NC, N_EDGES))
    dsti = dstf.reshape(NC * N_EDGES // K, K)
    dstf = dstf.reshape(NC * N_EDGES)
    w_cat = _linear_relu_split(x, Wn, bn.reshape(NC, HALF), N_NODES, 10000)
    h_cat = _linear_relu_split(edge_attr, We, be.reshape(NC, HALF), N_EDGES, 16000)
    return _sc_call(w_cat, h_cat, srci, dsti, dstf)
